# baseline (speedup 1.0000x reference)
import functools

import jax
import jax.numpy as jnp
from jax import lax
from jax.experimental import pallas as pl
from jax.experimental.pallas import tpu as pltpu
from jax.experimental.pallas import tpu_sc as plsc

N_NODES = 10000
D = 128
NC = 2
NS = 16
NW = NC * NS
NP = 10240
NPS = NP // NS
CHUNK = 128


def _cdiv(a, b):
    return (a + b - 1) // b


def _tc_prep_body(x_ref, w_ref, as_ref, ad_ref, h_ref, a_ref, b_ref, u_ref):
    h = jnp.dot(x_ref[...], w_ref[...], preferred_element_type=jnp.float32)
    h_ref[...] = h
    a = jnp.sum(h * as_ref[...], axis=1)
    b = jnp.sum(h * ad_ref[...], axis=1)
    a_ref[...] = a
    b_ref[...] = b
    u_ref[...] = jnp.full((16,), jnp.max(a) + jnp.max(b), jnp.float32)


def _tc_prep(x_p, W, att_src, att_dst):
    return pl.pallas_call(
        _tc_prep_body,
        out_shape=(
            jax.ShapeDtypeStruct((NP, D), jnp.float32),
            jax.ShapeDtypeStruct((NP,), jnp.float32),
            jax.ShapeDtypeStruct((NP,), jnp.float32),
            jax.ShapeDtypeStruct((16,), jnp.float32),
        ),
    )(x_p, W, att_src, att_dst)


def _tc_finish_body(p_ref, bias_ref, gamma_ref, beta_ref, o_ref):
    o = p_ref[0] + p_ref[1] + bias_ref[...]
    o = jnp.maximum(o, 0.0)
    scale = gamma_ref[...] / jnp.sqrt(jnp.float32(1.0 + 1e-5))
    o_ref[...] = o * scale + beta_ref[...]


def _tc_finish(outp, bias, gamma, beta):
    return pl.pallas_call(
        _tc_finish_body,
        out_shape=jax.ShapeDtypeStruct((NP, D), jnp.float32),
    )(outp, bias, gamma, beta)


@functools.lru_cache(maxsize=None)
def _make_sc_alpha(T, E_TOT):
    mesh = plsc.VectorSubcoreMesh(core_axis_name="c", subcore_axis_name="s", num_cores=NC, num_subcores=NS)
    EP = NW * T

    @functools.partial(
        pl.kernel,
        out_type=(
            jax.ShapeDtypeStruct((EP,), jnp.float32),
            jax.ShapeDtypeStruct((NC, NP), jnp.float32),
        ),
        mesh=mesh,
        compiler_params=pltpu.CompilerParams(needs_layout_passes=False),
        scratch_types=[
            pltpu.VMEM((NP,), jnp.float32),
            pltpu.VMEM((NP,), jnp.float32),
            pltpu.VMEM((T,), jnp.int32),
            pltpu.VMEM((T,), jnp.int32),
            pltpu.VMEM((T,), jnp.float32),
            pltpu.VMEM((NP,), jnp.float32),
            pltpu.VMEM((16,), jnp.float32),
            pltpu.VMEM((NS, NPS), jnp.float32),
            pltpu.VMEM_SHARED((NS, NP), jnp.float32),
        ],
    )
    def sc_alpha(se_hbm, de_hbm, asrc_hbm, adst_hbm, u_hbm,
                 ex_hbm, den_hbm,
                 asrc_v, adst_v, se_v, de_v, ex_v, den_v, u_v, red_v, den_sh):
        cid = lax.axis_index("c")
        sid = lax.axis_index("s")
        wid = sid * NC + cid
        base = wid * T
        pltpu.sync_copy(asrc_hbm, asrc_v)
        pltpu.sync_copy(adst_hbm, adst_v)
        pltpu.sync_copy(se_hbm.at[pl.ds(base, T)], se_v)
        pltpu.sync_copy(de_hbm.at[pl.ds(base, T)], de_v)
        pltpu.sync_copy(u_hbm, u_v)
        uvec = plsc.load_gather(u_v, [jnp.zeros((16,), jnp.int32)])
        zeros16 = jnp.zeros((16,), jnp.float32)

        @pl.loop(0, NP // 16)
        def _zero(i):
            den_v[pl.ds(i * 16, 16)] = zeros16

        @pl.loop(0, T // 16)
        def _edges(i):
            e = i * 16
            sidx = se_v[pl.ds(e, 16)]
            didx = de_v[pl.ds(e, 16)]
            av = plsc.load_gather(asrc_v, [sidx])
            bv = plsc.load_gather(adst_v, [didx])
            al = av + bv
            al = jnp.where(al > 0, al, al * jnp.float32(0.2))
            exv = jnp.exp(al - uvec)
            eid = base + e + lax.iota(jnp.int32, 16)
            exv = jnp.where(eid < E_TOT, exv, jnp.float32(0.0))
            ex_v[pl.ds(e, 16)] = exv
            plsc.addupdate_scatter(den_v, [didx], exv)

        pltpu.sync_copy(ex_v, ex_hbm.at[pl.ds(base, T)])
        pltpu.sync_copy(den_v, den_sh.at[sid])
        plsc.subcore_barrier()
        col0 = sid * NPS
        pltpu.sync_copy(den_sh.at[:, pl.ds(col0, NPS)], red_v)

        @pl.loop(0, NPS // 16)
        def _red(ci):
            c = ci * 16
            acc = red_v[0, pl.ds(c, 16)]
            for r in range(1, NS):
                acc = acc + red_v[r, pl.ds(c, 16)]
            den_v[pl.ds(c, 16)] = acc

        pltpu.sync_copy(den_v.at[pl.ds(0, NPS)], den_hbm.at[cid, pl.ds(col0, NPS)])

    return sc_alpha


@functools.lru_cache(maxsize=None)
def _make_sc_agg(T):
    mesh = plsc.VectorSubcoreMesh(core_axis_name="c", subcore_axis_name="s", num_cores=NC, num_subcores=NS)
    NCH = T // CHUNK
    NPAIR = NCH // 2

    @functools.partial(
        pl.kernel,
        out_type=jax.ShapeDtypeStruct((NC, NP, D), jnp.float32),
        mesh=mesh,
        compiler_params=pltpu.CompilerParams(needs_layout_passes=False),
        scratch_types=[
            pltpu.VMEM((2, 2, CHUNK), jnp.int32),
            pltpu.VMEM((2, 2, CHUNK), jnp.int32),
            pltpu.VMEM((2, CHUNK), jnp.float32),
            pltpu.VMEM((2, CHUNK), jnp.float32),
            pltpu.VMEM((CHUNK,), jnp.int32),
            pltpu.VMEM((CHUNK,), jnp.int32),
            pltpu.VMEM((CHUNK,), jnp.float32),
            pltpu.VMEM((CHUNK,), jnp.float32),
            pltpu.VMEM((CHUNK, D), jnp.float32),
            pltpu.VMEM((CHUNK, D), jnp.float32),
            pltpu.VMEM((NP,), jnp.float32),
            pltpu.VMEM((2048,), jnp.float32),
            pltpu.VMEM_SHARED((NP, D), jnp.float32),
            pltpu.SemaphoreType.DMA,
            pltpu.SemaphoreType.DMA,
            pltpu.SemaphoreType.DMA,
            pltpu.SemaphoreType.DMA,
            pltpu.SemaphoreType.DMA,
            pltpu.SemaphoreType.DMA,
        ],
    )
    def sc_agg(pk_hbm, ex2_hbm, den_hbm, h_hbm,
               out_hbm,
               P0, P1, E0, E1, de_s0, de_s1, co0, co1, r0, r1, d_v, d2s,
               acc_sh, sem_pk0, sem_pk1, sem_g0, sem_g1, sem_s0, sem_s1):
        cid = lax.axis_index("c")
        sid = lax.axis_index("s")
        wid = sid * NC + cid
        zeros16 = jnp.zeros((16,), jnp.float32)
        P = [P0, P1]
        E = [E0, E1]
        DE = [de_s0, de_s1]
        CO = [co0, co1]
        R = [r0, r1]
        SPK = [sem_pk0, sem_pk1]
        SG = [sem_g0, sem_g1]
        SS = [sem_s0, sem_s1]

        pltpu.sync_copy(den_hbm.at[0], d_v)

        @pl.loop(0, NP // 2048)
        def _dsum(b):
            pltpu.sync_copy(den_hbm.at[1, pl.ds(b * 2048, 2048)], d2s)

            @pl.loop(0, 2048 // 16)
            def _dadd(i):
                o = i * 16
                d_v[pl.ds(b * 2048 + o, 16)] = (
                    d_v[pl.ds(b * 2048 + o, 16)] + d2s[pl.ds(o, 16)])

        @pl.loop(0, CHUNK)
        def _zr(r):
            for c in range(D // 16):
                r0[r, pl.ds(c * 16, 16)] = zeros16

        @pl.loop(0, NPS // CHUNK)
        def _zacc(b):
            pltpu.sync_copy(r0, acc_sh.at[pl.ds(sid * NPS + b * CHUNK, CHUNK)])

        plsc.subcore_barrier()

        def issue_pair(g, p):
            pltpu.async_copy(pk_hbm.at[wid, g], P[p], SPK[p])
            pltpu.async_copy(ex2_hbm.at[wid, g], E[p], SPK[p])

        def wait_pair(g, p):
            pltpu.make_async_copy(pk_hbm.at[wid, g], P[p], SPK[p]).wait()
            pltpu.make_async_copy(ex2_hbm.at[wid, g], E[p], SPK[p]).wait()

        def issue_gather(p, slot, b):
            pltpu.async_copy(h_hbm.at[P[p].at[slot, 0]], R[b], SG[b])

        def wait_gather(p, slot, b):
            pltpu.make_async_copy(h_hbm.at[P[p].at[slot, 0]], R[b], SG[b]).wait()

        def issue_scatter(b):
            pltpu.async_copy(R[b], acc_sh.at[DE[b]], SS[b], add=True)

        def wait_scatter(b):
            pltpu.make_async_copy(R[b], acc_sh.at[DE[b]], SS[b]).wait()

        def chunk(g, p, slot, b,
                  do_wait_prev_scatter=True,
                  refill_pair=None,
                  wait_next_pair=False,
                  next_gather=None):
            for i in range(CHUNK // 16):
                e = i * 16
                didx = P[p][slot, 1, pl.ds(e, 16)]
                DE[b][pl.ds(e, 16)] = didx
                dv = plsc.load_gather(d_v, [didx])
                CO[b][pl.ds(e, 16)] = (
                    E[p][slot, pl.ds(e, 16)] / (dv + jnp.float32(1e-16)))
            wait_gather(p, slot, b)
            if refill_pair is not None:
                issue_pair(refill_pair, p)
            if do_wait_prev_scatter:
                wait_scatter(1 - b)
            if wait_next_pair:
                wait_pair_g, wait_pair_p = wait_next_pair
                wait_pair(wait_pair_g, wait_pair_p)
            if next_gather is not None:
                p2, slot2 = next_gather
                issue_gather(p2, slot2, 1 - b)

            @pl.loop(0, CHUNK, unroll=4)
            def _scale(r):
                cj = plsc.load_gather(CO[b], [jnp.full((16,), r, jnp.int32)])
                for c in range(D // 16):
                    R[b][r, pl.ds(c * 16, 16)] = (
                        R[b][r, pl.ds(c * 16, 16)] * cj)

            issue_scatter(b)

        issue_pair(0, 0)
        issue_pair(1, 1)
        wait_pair(0, 0)
        issue_gather(0, 0, 0)

        chunk(0, 0, 0, 0, do_wait_prev_scatter=False,
              next_gather=(0, 1))
        chunk(1, 0, 1, 1, refill_pair=jnp.int32(2),
              wait_next_pair=(jnp.int32(1), 1), next_gather=(1, 0))

        @pl.loop(1, NPAIR - 2)
        def _pair(g):
            pg = g
            @pl.when(g % 2 == 1)
            def _():
                chunk(g, 1, 0, 0, next_gather=(1, 1))
                chunk(g, 1, 1, 1, refill_pair=g + 2,
                      wait_next_pair=(g + 1, 0), next_gather=(0, 0))

            @pl.when(g % 2 == 0)
            def _():
                chunk(g, 0, 0, 0, next_gather=(0, 1))
                chunk(g, 0, 1, 1, refill_pair=g + 2,
                      wait_next_pair=(g + 1, 1), next_gather=(1, 0))

        gp = (NPAIR - 2) % 2
        chunk(NPAIR - 2, gp, 0, 0, next_gather=(gp, 1))
        chunk(NPAIR - 2, gp, 1, 1,
              wait_next_pair=(jnp.int32(NPAIR - 1), 1 - gp),
              next_gather=(1 - gp, 0))

        gl = (NPAIR - 1) % 2
        chunk(NPAIR - 1, gl, 0, 0, next_gather=(gl, 1))
        chunk(NPAIR - 1, gl, 1, 1, next_gather=None)
        wait_scatter(1)

        plsc.subcore_barrier()
        row0 = sid * NPS
        pltpu.sync_copy(acc_sh.at[pl.ds(row0, NPS)],
                        out_hbm.at[cid, pl.ds(row0, NPS)])

    return sc_agg


def kernel(x, edge_index, W, att_src, att_dst, bias, gamma, beta):
    N = x.shape[0]
    E = edge_index.shape[1]
    E_TOT = E + N
    NCH = 2 * _cdiv(E_TOT, NW * CHUNK * 2)
    T = NCH * CHUNK
    EP = NW * T
    PAD = EP - E_TOT

    loops = jnp.arange(N, dtype=jnp.int32)
    src = jnp.concatenate([
        edge_index[0].astype(jnp.int32), loops,
        jnp.zeros((PAD,), jnp.int32)])
    dst = jnp.concatenate([
        edge_index[1].astype(jnp.int32), loops,
        jnp.full((PAD,), N, jnp.int32)])

    x_p = jnp.pad(x, ((0, NP - N), (0, 0)))
    h, a_src_n, a_dst_n, u = _tc_prep(
        x_p, W, att_src.reshape(1, D), att_dst.reshape(1, D))

    ex, den = _make_sc_alpha(T, E_TOT)(src, dst, a_src_n, a_dst_n, u)

    pk = jnp.stack([src.reshape(NW, NCH, CHUNK), dst.reshape(NW, NCH, CHUNK)],
                   axis=2).reshape(NW, NCH // 2, 2, 2, CHUNK)
    outp = _make_sc_agg(T)(
        pk, ex.reshape(NW, NCH // 2, 2, CHUNK), den, h)

    out_full = _tc_finish(outp, bias.reshape(1, D), gamma.reshape(1, D),
                          beta.reshape(1, D))
    return out_full[:N]


# speedup vs baseline: 1.4466x; 1.4466x over previous
import functools

import jax
import jax.numpy as jnp
from jax import lax
from jax.experimental import pallas as pl
from jax.experimental.pallas import tpu as pltpu
from jax.experimental.pallas import tpu_sc as plsc

N_NODES = 10000
D = 128
NC = 2
NS = 16
NW = NC * NS
NP = 10240
NPS = NP // NS
CHUNK = 128


def _cdiv(a, b):
    return (a + b - 1) // b


def _tc_prep_body(x_ref, w_ref, as_ref, ad_ref, h_ref, a_ref, b_ref, u_ref):
    h = jnp.dot(x_ref[...], w_ref[...], preferred_element_type=jnp.float32)
    h_ref[...] = h
    a = jnp.sum(h * as_ref[...], axis=1)
    b = jnp.sum(h * ad_ref[...], axis=1)
    a_ref[...] = a
    b_ref[...] = b
    u_ref[...] = jnp.full((16,), jnp.max(a) + jnp.max(b), jnp.float32)


def _tc_prep(x_p, W, att_src, att_dst):
    return pl.pallas_call(
        _tc_prep_body,
        out_shape=(
            jax.ShapeDtypeStruct((NP, D), jnp.float32),
            jax.ShapeDtypeStruct((NP,), jnp.float32),
            jax.ShapeDtypeStruct((NP,), jnp.float32),
            jax.ShapeDtypeStruct((16,), jnp.float32),
        ),
    )(x_p, W, att_src, att_dst)


def _tc_finish_body(p_ref, bias_ref, gamma_ref, beta_ref, o_ref):
    o = p_ref[0] + p_ref[1] + bias_ref[...]
    o = jnp.maximum(o, 0.0)
    scale = gamma_ref[...] / jnp.sqrt(jnp.float32(1.0 + 1e-5))
    o_ref[...] = o * scale + beta_ref[...]


def _tc_finish(outp, bias, gamma, beta):
    return pl.pallas_call(
        _tc_finish_body,
        out_shape=jax.ShapeDtypeStruct((NP, D), jnp.float32),
    )(outp, bias, gamma, beta)


@functools.lru_cache(maxsize=None)
def _make_sc_alpha(T, E_TOT):
    mesh = plsc.VectorSubcoreMesh(core_axis_name="c", subcore_axis_name="s", num_cores=NC, num_subcores=NS)
    EP = NW * T

    @functools.partial(
        pl.kernel,
        out_type=(
            jax.ShapeDtypeStruct((EP,), jnp.float32),
            jax.ShapeDtypeStruct((NC, NP), jnp.float32),
        ),
        mesh=mesh,
        compiler_params=pltpu.CompilerParams(needs_layout_passes=False),
        scratch_types=[
            pltpu.VMEM((NP,), jnp.float32),
            pltpu.VMEM((NP,), jnp.float32),
            pltpu.VMEM((T,), jnp.int32),
            pltpu.VMEM((T,), jnp.int32),
            pltpu.VMEM((T,), jnp.float32),
            pltpu.VMEM((NP,), jnp.float32),
            pltpu.VMEM((16,), jnp.float32),
            pltpu.VMEM((NS, NPS), jnp.float32),
            pltpu.VMEM_SHARED((NS, NP), jnp.float32),
        ],
    )
    def sc_alpha(se_hbm, de_hbm, asrc_hbm, adst_hbm, u_hbm,
                 ex_hbm, den_hbm,
                 asrc_v, adst_v, se_v, de_v, ex_v, den_v, u_v, red_v, den_sh):
        cid = lax.axis_index("c")
        sid = lax.axis_index("s")
        wid = sid * NC + cid
        base = wid * T
        pltpu.sync_copy(asrc_hbm, asrc_v)
        pltpu.sync_copy(adst_hbm, adst_v)
        pltpu.sync_copy(se_hbm.at[pl.ds(base, T)], se_v)
        pltpu.sync_copy(de_hbm.at[pl.ds(base, T)], de_v)
        pltpu.sync_copy(u_hbm, u_v)
        uvec = plsc.load_gather(u_v, [jnp.zeros((16,), jnp.int32)])
        zeros16 = jnp.zeros((16,), jnp.float32)

        @pl.loop(0, NP // 16)
        def _zero(i):
            den_v[pl.ds(i * 16, 16)] = zeros16

        @pl.loop(0, T // 16)
        def _edges(i):
            e = i * 16
            sidx = se_v[pl.ds(e, 16)]
            didx = de_v[pl.ds(e, 16)]
            av = plsc.load_gather(asrc_v, [sidx])
            bv = plsc.load_gather(adst_v, [didx])
            al = av + bv
            al = jnp.where(al > 0, al, al * jnp.float32(0.2))
            exv = jnp.exp(al - uvec)
            eid = base + e + lax.iota(jnp.int32, 16)
            exv = jnp.where(eid < E_TOT, exv, jnp.float32(0.0))
            ex_v[pl.ds(e, 16)] = exv
            plsc.addupdate_scatter(den_v, [didx], exv)

        pltpu.sync_copy(ex_v, ex_hbm.at[pl.ds(base, T)])
        pltpu.sync_copy(den_v, den_sh.at[sid])
        plsc.subcore_barrier()
        col0 = sid * NPS
        pltpu.sync_copy(den_sh.at[:, pl.ds(col0, NPS)], red_v)

        @pl.loop(0, NPS // 16)
        def _red(ci):
            c = ci * 16
            acc = red_v[0, pl.ds(c, 16)]
            for r in range(1, NS):
                acc = acc + red_v[r, pl.ds(c, 16)]
            den_v[pl.ds(c, 16)] = acc

        pltpu.sync_copy(den_v.at[pl.ds(0, NPS)], den_hbm.at[cid, pl.ds(col0, NPS)])

    return sc_alpha


@functools.lru_cache(maxsize=None)
def _make_sc_agg(T):
    mesh = plsc.VectorSubcoreMesh(core_axis_name="c", subcore_axis_name="s", num_cores=NC, num_subcores=NS)
    NCH = T // CHUNK

    @functools.partial(
        pl.kernel,
        out_type=jax.ShapeDtypeStruct((NC, NP, D), jnp.float32),
        mesh=mesh,
        compiler_params=pltpu.CompilerParams(needs_layout_passes=False),
        scratch_types=[
            pltpu.VMEM((2, CHUNK), jnp.int32),
            pltpu.VMEM((2, CHUNK), jnp.int32),
            pltpu.VMEM((CHUNK,), jnp.int32),
            pltpu.VMEM((CHUNK,), jnp.int32),
            pltpu.VMEM((CHUNK,), jnp.float32),
            pltpu.VMEM((CHUNK,), jnp.float32),
            pltpu.VMEM((CHUNK, D), jnp.float32),
            pltpu.VMEM((CHUNK, D), jnp.float32),
            pltpu.VMEM((NP,), jnp.float32),
            pltpu.VMEM((2048,), jnp.float32),
            pltpu.VMEM_SHARED((NP, D), jnp.float32),
            pltpu.SemaphoreType.DMA,
            pltpu.SemaphoreType.DMA,
            pltpu.SemaphoreType.DMA,
            pltpu.SemaphoreType.DMA,
            pltpu.SemaphoreType.DMA,
            pltpu.SemaphoreType.DMA,
            pltpu.SemaphoreType.DMA,
            pltpu.SemaphoreType.DMA,
        ],
    )
    def sc_agg(pk2_hbm, ex2_hbm, den_hbm, h_hbm,
               out_hbm,
               pk0, pk1, ds0, ds1, ex0, ex1, r0, r1, d_v, d2s, acc_sh,
               sem_pk0, sem_pk1, sem_ex0, sem_ex1,
               sem_g0, sem_g1, sem_s0, sem_s1):
        cid = lax.axis_index("c")
        sid = lax.axis_index("s")
        wid = sid * NC + cid
        zeros16 = jnp.zeros((16,), jnp.float32)
        bufs = [
            (pk0, ds0, ex0, r0, sem_pk0, sem_ex0, sem_g0, sem_s0),
            (pk1, ds1, ex1, r1, sem_pk1, sem_ex1, sem_g1, sem_s1),
        ]

        pltpu.sync_copy(den_hbm.at[0], d_v)

        @pl.loop(0, NP // 2048)
        def _dsum(b):
            pltpu.sync_copy(den_hbm.at[1, pl.ds(b * 2048, 2048)], d2s)

            @pl.loop(0, 2048 // 16)
            def _dadd(i):
                o = i * 16
                d_v[pl.ds(b * 2048 + o, 16)] = (
                    d_v[pl.ds(b * 2048 + o, 16)] + d2s[pl.ds(o, 16)])

        @pl.loop(0, CHUNK)
        def _zr(r):
            for c in range(D // 16):
                r0[r, pl.ds(c * 16, 16)] = zeros16

        @pl.loop(0, NPS // CHUNK)
        def _zacc(b):
            pltpu.sync_copy(r0, acc_sh.at[pl.ds(sid * NPS + b * CHUNK, CHUNK)])

        plsc.subcore_barrier()

        def body(j, cur, nxt):
            cpk, cds, cex, cr, csem_pk, csem_ex, csem_g, csem_s = cur
            npk, nds, nex, nr, nsem_pk, nsem_ex, nsem_g, nsem_s = nxt

            pltpu.make_async_copy(ex2_hbm.at[wid, j], cex, csem_ex).wait()

            @pl.loop(0, CHUNK // 16, unroll=2)
            def _coef(i):
                e = i * 16
                didx = cpk[1, pl.ds(e, 16)]
                cds[pl.ds(e, 16)] = didx
                dv = plsc.load_gather(d_v, [didx])
                cex[pl.ds(e, 16)] = (
                    cex[pl.ds(e, 16)] / (dv + jnp.float32(1e-16)))

            pltpu.make_async_copy(h_hbm.at[cpk.at[0]], cr, csem_g).wait()

            @pl.when(j + 2 < NCH)
            def _():
                pltpu.async_copy(pk2_hbm.at[wid, j + 2], cpk, csem_pk)

            @pl.when(j > 0)
            def _():
                pltpu.make_async_copy(nr, acc_sh.at[nds], nsem_s).wait()

            @pl.when(j + 1 < NCH)
            def _():
                pltpu.make_async_copy(
                    pk2_hbm.at[wid, j + 1], npk, nsem_pk).wait()
                pltpu.async_copy(h_hbm.at[npk.at[0]], nr, nsem_g)
                pltpu.async_copy(ex2_hbm.at[wid, j + 1], nex, nsem_ex)

            @pl.loop(0, CHUNK, unroll=4)
            def _scale(r):
                cj = plsc.load_gather(cex, [jnp.full((16,), r, jnp.int32)])
                for c in range(D // 16):
                    cr[r, pl.ds(c * 16, 16)] = cr[r, pl.ds(c * 16, 16)] * cj

            pltpu.async_copy(cr, acc_sh.at[cds], csem_s, add=True)

        pltpu.async_copy(pk2_hbm.at[wid, 0], pk0, sem_pk0)
        pltpu.async_copy(ex2_hbm.at[wid, 0], ex0, sem_ex0)
        pltpu.async_copy(pk2_hbm.at[wid, 1], pk1, sem_pk1)
        pltpu.make_async_copy(pk2_hbm.at[wid, 0], pk0, sem_pk0).wait()
        pltpu.async_copy(h_hbm.at[pk0.at[0]], r0, sem_g0)

        @pl.loop(0, NCH)
        def _chunk(j):
            @pl.when(j % 2 == 0)
            def _():
                body(j, bufs[0], bufs[1])

            @pl.when(j % 2 == 1)
            def _():
                body(j, bufs[1], bufs[0])

        _, lds, _, lr, _, _, _, lsem_s = bufs[(NCH - 1) % 2]
        pltpu.make_async_copy(lr, acc_sh.at[lds], lsem_s).wait()

        plsc.subcore_barrier()
        row0 = sid * NPS
        pltpu.sync_copy(acc_sh.at[pl.ds(row0, NPS)],
                        out_hbm.at[cid, pl.ds(row0, NPS)])

    return sc_agg


def kernel(x, edge_index, W, att_src, att_dst, bias, gamma, beta):
    N = x.shape[0]
    E = edge_index.shape[1]
    E_TOT = E + N
    NCH = _cdiv(E_TOT, NW * CHUNK)
    T = NCH * CHUNK
    EP = NW * T
    PAD = EP - E_TOT

    loops = jnp.arange(N, dtype=jnp.int32)
    src = jnp.concatenate([
        edge_index[0].astype(jnp.int32), loops,
        jnp.zeros((PAD,), jnp.int32)])
    dst = jnp.concatenate([
        edge_index[1].astype(jnp.int32), loops,
        jnp.full((PAD,), N, jnp.int32)])

    x_p = jnp.pad(x, ((0, NP - N), (0, 0)))
    h, a_src_n, a_dst_n, u = _tc_prep(
        x_p, W, att_src.reshape(1, D), att_dst.reshape(1, D))

    ex, den = _make_sc_alpha(T, E_TOT)(src, dst, a_src_n, a_dst_n, u)

    pk = jnp.stack(
        [src.reshape(NW, NCH, CHUNK), dst.reshape(NW, NCH, CHUNK)], axis=2)
    outp = _make_sc_agg(T)(pk, ex.reshape(NW, NCH, CHUNK), den, h)

    out_full = _tc_finish(outp, bias.reshape(1, D), gamma.reshape(1, D),
                          beta.reshape(1, D))
    return out_full[:N]
